# baseline (speedup 1.0000x reference)
import functools

import jax
import jax.numpy as jnp
from jax import lax
from jax.experimental import pallas as pl
from jax.experimental.pallas import tpu as pltpu
from jax.experimental.pallas import tpu_sc as plsc

N = 10000
E = 320000
HID = 128
NLVL = 8

NC = 2
NS = 16
NW = NC * NS
EPW = E // NW
NGRP = EPW // 16
CHUNK = 128
DCH = 80
NKCH = EPW // DCH
WAVE = 25
NP = 10240
RPS = NP // NS
ZU = 64
TRASH = NP - 8
EP = E + CHUNK * 256
STAG = EPW + CHUNK
NBUF = 2

_mesh = plsc.VectorSubcoreMesh(
    core_axis_name="c", subcore_axis_name="s", num_cores=NC, num_subcores=NS)
_params = pltpu.CompilerParams(needs_layout_passes=False)


def _iota16():
    return lax.iota(jnp.int32, 16)


def _lane(oh, v16):
    return jnp.sum(jnp.where(oh == 1, v16, 0))


@functools.partial(
    pl.kernel,
    out_type=[jax.ShapeDtypeStruct((NC, NP, HID), jnp.float32),
              jax.ShapeDtypeStruct((NW, 8, 128), jnp.int32)],
    mesh=_mesh,
    compiler_params=_params,
    scratch_types=[
        pltpu.VMEM((NKCH, DCH), jnp.int32),
        pltpu.VMEM((DCH, HID), jnp.float32),
        pltpu.VMEM((ZU, HID), jnp.float32),
        pltpu.VMEM((N,), jnp.int32),
        pltpu.VMEM((8, 128), jnp.int32),
        pltpu.VMEM_SHARED((NP, HID), jnp.float32),
        pltpu.SemaphoreType.DMA,
        pltpu.SemaphoreType.DMA,
    ],
)
def _sc_degp1(dst2_hbm, lvl_hbm, ones_hbm, zrs_hbm, deg_hbm, cnt_hbm,
              dstblk, ones_v, zero_v, lvl_v, cblk, acc_sh, sem, dsem):
    c = lax.axis_index("c")
    s = lax.axis_index("s")
    wid = s * NC + c
    pltpu.sync_copy(dst2_hbm.at[wid], dstblk)
    pltpu.sync_copy(lvl_hbm, lvl_v)
    pltpu.sync_copy(ones_hbm, ones_v)
    pltpu.sync_copy(zrs_hbm, zero_v)

    for r in range(8):
        for q in range(8):
            cblk[r, pl.ds(q * 16, 16)] = jnp.zeros((16,), jnp.int32)

    def _zcpy(i, _):
        off = pl.multiple_of(s * RPS + i * ZU, 8)
        pltpu.sync_copy(zero_v, acc_sh.at[pl.ds(off, ZU)])
        return 0
    lax.fori_loop(0, RPS // ZU, _zcpy, 0)
    plsc.subcore_barrier()

    def _wave(wv, cnt):
        def _fire(k, _):
            pltpu.async_copy(ones_v, acc_sh.at[dstblk.at[wv * WAVE + k]],
                             dsem, add=True)
            return 0
        lax.fori_loop(0, WAVE, _fire, 0)

        def _grp(g, cn):
            g2 = wv * (NGRP // 5) + g
            d16 = dstblk[g2 // 5, pl.ds((g2 % 5) * 16, 16)]
            dlev = plsc.load_gather(lvl_v, [d16])
            for l in range(1, 8):
                pc = plsc.all_reduce_population_count(dlev == l)
                cn = cn + jnp.where(_iota16() == l, pc, 0)
            return cn
        cnt = lax.fori_loop(0, NGRP // 5, _grp, cnt)

        def _drain(k, _):
            pltpu.make_async_copy(ones_v, acc_sh.at[dstblk.at[0]],
                                  dsem).wait()
            return 0
        lax.fori_loop(0, WAVE, _drain, 0)
        return cnt
    cnt = lax.fori_loop(0, NKCH // WAVE, _wave, jnp.zeros((16,), jnp.int32))

    cblk[0, pl.ds(0, 16)] = cnt
    pltpu.sync_copy(cblk, cnt_hbm.at[wid])
    plsc.subcore_barrier()

    off = pl.multiple_of(s * RPS, 8)
    pltpu.sync_copy(acc_sh.at[pl.ds(off, RPS)],
                    deg_hbm.at[c, pl.ds(off, RPS)])


def _bucket_layout(cnts_v, wid):
    total = jnp.zeros((16,), jnp.int32)
    mine = jnp.zeros((16,), jnp.int32)
    widv = jnp.zeros((16,), jnp.int32) + wid
    for wo in range(NW):
        row = cnts_v[wo, 0, pl.ds(0, 16)]
        pcw = ((row + (CHUNK - 1)) // CHUNK) * CHUNK
        total = total + pcw
        mine = mine + jnp.where(widv > wo, pcw, 0)
    base = plsc.cumsum(total) - total
    return base, total, mine


@functools.partial(
    pl.kernel,
    out_type=[jax.ShapeDtypeStruct((EP,), jnp.int32),
              jax.ShapeDtypeStruct((EP,), jnp.int32),
              jax.ShapeDtypeStruct((8, 128), jnp.int32)],
    mesh=_mesh,
    compiler_params=_params,
    scratch_types=[
        pltpu.VMEM((EPW,), jnp.int32),
        pltpu.VMEM((EPW,), jnp.int32),
        pltpu.VMEM((EPW,), jnp.int32),
        pltpu.VMEM((N,), jnp.int32),
        pltpu.VMEM((NW, 8, 128), jnp.int32),
        pltpu.VMEM((STAG,), jnp.int32),
        pltpu.VMEM((STAG,), jnp.int32),
        pltpu.VMEM((8, 128), jnp.int32),
        pltpu.SemaphoreType.DMA,
        pltpu.SemaphoreType.DMA,
    ],
)
def _sc_pass2(src_hbm, dst_hbm, lvl_hbm, cnt_hbm, srcc_hbm, dstc_hbm,
              summ_hbm, srcblk, dstblk, dlev_v, lvl_v, cnts_v, sstag, dstag,
              sblk, sem, osem):
    c = lax.axis_index("c")
    s = lax.axis_index("s")
    wid = s * NC + c
    woff = pl.multiple_of(wid * EPW, 8)
    pltpu.sync_copy(src_hbm.at[pl.ds(woff, EPW)], srcblk)
    pltpu.sync_copy(dst_hbm.at[pl.ds(woff, EPW)], dstblk)
    pltpu.sync_copy(lvl_hbm, lvl_v)
    pltpu.sync_copy(cnt_hbm, cnts_v)

    def _pg(g, _):
        d16 = dstblk[pl.ds(g * 16, 16)]
        dlev_v[pl.ds(g * 16, 16)] = plsc.load_gather(lvl_v, [d16])
        return 0
    lax.fori_loop(0, NGRP, _pg, 0)

    base, total, mine = _bucket_layout(cnts_v, wid)
    slot = base + mine

    @pl.when(wid == 0)
    def _pub():
        for r in range(8):
            for q in range(8):
                sblk[r, pl.ds(q * 16, 16)] = jnp.zeros((16,), jnp.int32)
        sblk[0, pl.ds(0, 16)] = base
        sblk[1, pl.ds(0, 16)] = total

        def _ng(g, nc):
            lv16 = lvl_v[pl.ds(g * 16, 16)]
            for l in range(8):
                pc = plsc.all_reduce_population_count(lv16 == l)
                nc = nc + jnp.where(_iota16() == l, pc, 0)
            return nc
        ncnt = lax.fori_loop(0, N // 16, _ng, jnp.zeros((16,), jnp.int32))
        sblk[2, pl.ds(0, 16)] = plsc.cumsum(ncnt) - ncnt
        sblk[3, pl.ds(0, 16)] = ncnt
        pltpu.sync_copy(sblk, summ_hbm)

    for l in range(1, 8):
        def _grp(g, ptr):
            s16 = srcblk[pl.ds(g * 16, 16)]
            d16 = dstblk[pl.ds(g * 16, 16)]
            m = dlev_v[pl.ds(g * 16, 16)] == l
            mi = m.astype(jnp.int32)
            idx = (plsc.cumsum(mi) - mi) + ptr
            plsc.store_scatter(sstag, [idx], s16, mask=m)
            plsc.store_scatter(dstag, [idx], d16, mask=m)
            return ptr + jnp.sum(mi)
        ptr = lax.fori_loop(0, NGRP, _grp, jnp.zeros((), jnp.int32))

        ones16 = jnp.zeros((16,), jnp.int32) + 1
        for g in range(CHUNK // 16):
            pidx = _iota16() + (ptr + g * 16)
            plsc.store_scatter(sstag, [pidx], jnp.zeros((16,), jnp.int32),
                               mask=ones16 == 1)
            plsc.store_scatter(dstag, [pidx],
                               jnp.zeros((16,), jnp.int32) + TRASH,
                               mask=ones16 == 1)

        myslot = _lane((_iota16() == l).astype(jnp.int32), slot)
        nch = (ptr + CHUNK - 1) // CHUNK

        def _out(j, _):
            o = pl.multiple_of(j * CHUNK, 8)
            go = pl.multiple_of(myslot + o, 8)
            pltpu.async_copy(sstag.at[pl.ds(o, CHUNK)],
                             srcc_hbm.at[pl.ds(go, CHUNK)], osem)
            pltpu.async_copy(dstag.at[pl.ds(o, CHUNK)],
                             dstc_hbm.at[pl.ds(go, CHUNK)], osem)
            return 0
        lax.fori_loop(0, nch, _out, 0)

        def _dr(j, _):
            pltpu.make_async_copy(sstag.at[pl.ds(0, CHUNK)],
                                  srcc_hbm.at[pl.ds(0, CHUNK)], osem).wait()
            pltpu.make_async_copy(dstag.at[pl.ds(0, CHUNK)],
                                  dstc_hbm.at[pl.ds(0, CHUNK)], osem).wait()
            return 0
        lax.fori_loop(0, nch, _dr, 0)


def _make_agg(lv):
    @functools.partial(
        pl.kernel,
        out_type=jax.ShapeDtypeStruct((NC, NP, HID), jnp.float32),
        mesh=_mesh,
        compiler_params=_params,
        scratch_types=[
            pltpu.VMEM((8, 128), jnp.int32),
            pltpu.VMEM((CHUNK,), jnp.int32),
            pltpu.VMEM((CHUNK,), jnp.int32),
            pltpu.VMEM((CHUNK, HID), jnp.float32),
            pltpu.VMEM((ZU, HID), jnp.float32),
            pltpu.VMEM_SHARED((NP, HID), jnp.float32),
            pltpu.SemaphoreType.DMA,
        ],
    )
    def _agg(hext_hbm, srcc_hbm, dstc_hbm, summ_hbm, zrs_hbm, out_hbm,
             summ_v, src_v, dst_v, rows_v, zero_v, acc_sh, sem):
        c = lax.axis_index("c")
        s = lax.axis_index("s")
        wid = s * NC + c
        pltpu.sync_copy(summ_hbm, summ_v)
        pltpu.sync_copy(zrs_hbm, zero_v)
        oh = (_iota16() == lv).astype(jnp.int32)

        base_l = _lane(oh, summ_v[0, pl.ds(0, 16)])
        nch_l = _lane(oh, summ_v[1, pl.ds(0, 16)]) // CHUNK
        nb_l = _lane(oh, summ_v[2, pl.ds(0, 16)])
        ncn_l = _lane(oh, summ_v[3, pl.ds(0, 16)])
        T = (nch_l - wid + (NW - 1)) // NW

        a0 = (nb_l // 8) * 8
        e0 = ((nb_l + ncn_l + 7) // 8) * 8
        nu = (e0 - a0 + ZU - 1) // ZU
        zt = (nu - s + NS - 1) // NS

        def _zcpy(i, _):
            off = pl.multiple_of(a0 + (s + i * NS) * ZU, 8)
            pltpu.sync_copy(zero_v, acc_sh.at[pl.ds(off, ZU)])
            return 0
        lax.fori_loop(0, zt, _zcpy, 0)
        plsc.subcore_barrier()

        def _chunk(j, _):
            off = pl.multiple_of(base_l + (wid + j * NW) * CHUNK, 8)
            pltpu.sync_copy(srcc_hbm.at[pl.ds(off, CHUNK)], src_v)
            pltpu.sync_copy(dstc_hbm.at[pl.ds(off, CHUNK)], dst_v)
            pltpu.async_copy(hext_hbm.at[src_v], rows_v, sem).wait()
            pltpu.sync_copy(rows_v, acc_sh.at[dst_v], add=True)
            return 0
        lax.fori_loop(0, T, _chunk, 0)
        plsc.subcore_barrier()

        def _dcpy(i, _):
            off = pl.multiple_of(a0 + (s + i * NS) * ZU, 8)
            pltpu.sync_copy(acc_sh.at[pl.ds(off, ZU)],
                            out_hbm.at[c, pl.ds(off, ZU)])
            return 0
        lax.fori_loop(0, zt, _dcpy, 0)
    return _agg


_AGGS = {l: _make_agg(l) for l in range(1, NLVL)}


_RB = 1024
_NBLK = NP // _RB


def _leaky(x):
    return jnp.where(x >= 0, x, 0.1 * x)


def _tc_init_body(feat, delay, lvl, Wpi1, bpi1, Wpi2, bpi2, Ws1, bs1, Ws2,
                  bs2, h_out, hself_out):
    hs = jnp.dot(feat[...], Ws1[...], preferred_element_type=jnp.float32)
    hs = _leaky(hs + bs1[...])
    hs = jnp.dot(hs, Ws2[...], preferred_element_type=jnp.float32) + bs2[...]
    hself_out[...] = hs

    hp = delay[...] * Wpi1[...]
    hp = _leaky(hp + bpi1[...])
    hp = jnp.dot(hp, Wpi2[...], preferred_element_type=jnp.float32) + bpi2[...]
    h_out[...] = jnp.where(lvl[...] == 0, hp, 0.0)


def _tc_init(feat, delay, lvl2d, Wpi1, bpi1, Wpi2, bpi2, Ws1, bs1, Ws2, bs2):
    full = lambda shape: pl.BlockSpec(shape, lambda i: (0, 0))
    row = lambda w: pl.BlockSpec((_RB, w), lambda i: (i, 0))
    return pl.pallas_call(
        _tc_init_body,
        grid=(_NBLK,),
        in_specs=[row(HID), row(1), row(1),
                  full((1, 64)), full((1, 64)), full((64, HID)), full((1, HID)),
                  full((HID, 64)), full((1, 64)), full((64, HID)), full((1, HID))],
        out_specs=[row(HID), row(HID)],
        out_shape=[jax.ShapeDtypeStruct((NP, HID), jnp.float32),
                   jax.ShapeDtypeStruct((NP, HID), jnp.float32)],
    )(feat, delay, lvl2d, Wpi1, bpi1, Wpi2, bpi2, Ws1, bs1, Ws2, bs2)


def _tc_level_body(lref, accA, accB, degA, degB, hself, h_in, lvl,
                   ispo, Wn1, bn1, Wn2, bn2, h_out):
    lv = lref[0, 0]
    deg = jnp.maximum(degA[...] + degB[...], 1.0)
    neigh = (accA[...] + accB[...]) / deg
    hid = jnp.dot(neigh, Wn1[...], preferred_element_type=jnp.float32)
    hid = _leaky(hid + bn1[...])
    out = jnp.dot(hid, Wn2[...], preferred_element_type=jnp.float32) + bn2[...]
    out = out + hself[...]
    out = jnp.where(ispo[...] != 1, jnp.maximum(out, 0.0), out)
    h_out[...] = jnp.where(lvl[...] == lv, out, h_in[...])


def _tc_level(lval, accA, accB, degA, degB, hself, h, lvl2d, ispo,
              Wn1, bn1, Wn2, bn2):
    full = lambda shape: pl.BlockSpec(shape, lambda i: (0, 0))
    row = lambda w: pl.BlockSpec((_RB, w), lambda i: (i, 0))
    return pl.pallas_call(
        _tc_level_body,
        grid=(_NBLK,),
        in_specs=[pl.BlockSpec(memory_space=pltpu.SMEM),
                  row(HID), row(HID), row(1), row(1), row(HID), row(HID),
                  row(1), row(1),
                  full((HID, 64)), full((1, 64)), full((64, HID)), full((1, HID))],
        out_specs=row(HID),
        out_shape=jax.ShapeDtypeStruct((NP, HID), jnp.float32),
    )(lval, accA, accB, degA, degB, hself, h, lvl2d, ispo,
      Wn1, bn1, Wn2, bn2)


def kernel(feat, delay, is_po, edge_index, level_ids, Wpi1, bpi1, Wpi2, bpi2,
           Ws1, bs1, Ws2, bs2, Wn1, bn1, Wn2, bn2):
    src = edge_index[0]
    dst = edge_index[1]
    pad = NP - N
    featp = jnp.pad(feat, ((0, pad), (0, 0)))
    delayp = jnp.pad(delay, ((0, pad), (0, 0)))
    ispop = jnp.pad(is_po, ((0, pad), (0, 0)))
    lvlp = jnp.pad(level_ids, (0, pad), constant_values=99)[:, None]

    h, h_self = _tc_init(featp, delayp, lvlp,
                         Wpi1, bpi1[None, :], Wpi2, bpi2[None, :],
                         Ws1, bs1[None, :], Ws2, bs2[None, :])

    ones_tab = jnp.ones((DCH, HID), jnp.float32)
    zrs_tab = jnp.zeros((ZU, HID), jnp.float32)
    degp, cnts = _sc_degp1(dst.reshape(NW, NKCH, DCH), level_ids,
                           ones_tab, zrs_tab)
    degA = degp[0, :, 0:1]
    degB = degp[1, :, 0:1]
    srcc, dstc, summ = _sc_pass2(src, dst, level_ids, cnts)

    bn1r = bn1[None, :]
    bn2r = bn2[None, :]
    for l in range(1, NLVL):
        acc = _AGGS[l](h, srcc, dstc, summ, zrs_tab)
        lval = jnp.full((1, 1), l, dtype=jnp.int32)
        h = _tc_level(lval, acc[0], acc[1], degA, degB, h_self, h,
                      lvlp, ispop, Wn1, bn1r, Wn2, bn2r)
    return h[:N]


# speedup vs baseline: 1.3103x; 1.3103x over previous
import functools

import jax
import jax.numpy as jnp
from jax import lax
from jax.experimental import pallas as pl
from jax.experimental.pallas import tpu as pltpu
from jax.experimental.pallas import tpu_sc as plsc

N = 10000
E = 320000
HID = 128
NLVL = 8

NC = 2
NS = 16
NW = NC * NS
EPW = E // NW
NGRP = EPW // 16
CHUNK = 64
DCH = 80
NKCH = EPW // DCH
WAVE = 25
NP = 10240
RPS = NP // NS
ZU = 32
TRASH = NP - 8
EP = E + CHUNK * 256 + 8192
FAST = 32
KW = 2
STAG = EPW + CHUNK

_mesh = plsc.VectorSubcoreMesh(
    core_axis_name="c", subcore_axis_name="s", num_cores=NC, num_subcores=NS)
_params = pltpu.CompilerParams(needs_layout_passes=False)


def _iota16():
    return lax.iota(jnp.int32, 16)


def _lane(oh, v16):
    return jnp.sum(jnp.where(oh == 1, v16, 0))


@functools.partial(
    pl.kernel,
    out_type=[jax.ShapeDtypeStruct((NC, NP, HID), jnp.float32),
              jax.ShapeDtypeStruct((NW, 8, 128), jnp.int32)],
    mesh=_mesh,
    compiler_params=_params,
    scratch_types=[
        pltpu.VMEM((NKCH, DCH), jnp.int32),
        pltpu.VMEM((DCH, HID), jnp.float32),
        pltpu.VMEM((ZU, HID), jnp.float32),
        pltpu.VMEM((N,), jnp.int32),
        pltpu.VMEM((8, 128), jnp.int32),
        pltpu.VMEM_SHARED((NP, HID), jnp.float32),
        pltpu.SemaphoreType.DMA,
        pltpu.SemaphoreType.DMA,
    ],
)
def _sc_degp1(dst2_hbm, lvl_hbm, ones_hbm, zrs_hbm, deg_hbm, cnt_hbm,
              dstblk, ones_v, zero_v, lvl_v, cblk, acc_sh, sem, dsem):
    c = lax.axis_index("c")
    s = lax.axis_index("s")
    wid = s * NC + c
    pltpu.sync_copy(dst2_hbm.at[wid], dstblk)
    pltpu.sync_copy(lvl_hbm, lvl_v)
    pltpu.sync_copy(ones_hbm, ones_v)
    pltpu.sync_copy(zrs_hbm, zero_v)

    for r in range(8):
        for q in range(8):
            cblk[r, pl.ds(q * 16, 16)] = jnp.zeros((16,), jnp.int32)

    def _zcpy(i, _):
        off = pl.multiple_of(s * RPS + i * ZU, 8)
        pltpu.sync_copy(zero_v, acc_sh.at[pl.ds(off, ZU)])
        return 0
    lax.fori_loop(0, RPS // ZU, _zcpy, 0)
    plsc.subcore_barrier()

    def _wave(wv, cnt):
        def _fire(k, _):
            pltpu.async_copy(ones_v, acc_sh.at[dstblk.at[wv * WAVE + k]],
                             dsem, add=True)
            return 0
        lax.fori_loop(0, WAVE, _fire, 0)

        def _grp(g, cn):
            g2 = wv * (NGRP // 5) + g
            d16 = dstblk[g2 // 5, pl.ds((g2 % 5) * 16, 16)]
            dlev = plsc.load_gather(lvl_v, [d16])
            for l in range(1, 8):
                pc = plsc.all_reduce_population_count(dlev == l)
                cn = cn + jnp.where(_iota16() == l, pc, 0)
            return cn
        cnt = lax.fori_loop(0, NGRP // 5, _grp, cnt)

        def _drain(k, _):
            pltpu.make_async_copy(ones_v, acc_sh.at[dstblk.at[0]],
                                  dsem).wait()
            return 0
        lax.fori_loop(0, WAVE, _drain, 0)
        return cnt
    cnt = lax.fori_loop(0, NKCH // WAVE, _wave, jnp.zeros((16,), jnp.int32))

    cblk[0, pl.ds(0, 16)] = cnt
    pltpu.sync_copy(cblk, cnt_hbm.at[wid])
    plsc.subcore_barrier()

    off = pl.multiple_of(s * RPS, 8)
    pltpu.sync_copy(acc_sh.at[pl.ds(off, RPS)],
                    deg_hbm.at[c, pl.ds(off, RPS)])


def _bucket_layout(cnts_v, wid):
    total = jnp.zeros((16,), jnp.int32)
    mine = jnp.zeros((16,), jnp.int32)
    widv = jnp.zeros((16,), jnp.int32) + wid
    for wo in range(NW):
        row = cnts_v[wo, 0, pl.ds(0, 16)]
        pcw = ((row + (CHUNK - 1)) // CHUNK) * CHUNK
        total = total + pcw
        mine = mine + jnp.where(widv > wo, pcw, 0)
    base = plsc.cumsum(total) - total
    return base, total, mine


@functools.partial(
    pl.kernel,
    out_type=[jax.ShapeDtypeStruct((EP,), jnp.int32),
              jax.ShapeDtypeStruct((EP,), jnp.int32),
              jax.ShapeDtypeStruct((8, 128), jnp.int32)],
    mesh=_mesh,
    compiler_params=_params,
    scratch_types=[
        pltpu.VMEM((EPW,), jnp.int32),
        pltpu.VMEM((EPW,), jnp.int32),
        pltpu.VMEM((EPW,), jnp.int32),
        pltpu.VMEM((N,), jnp.int32),
        pltpu.VMEM((NW, 8, 128), jnp.int32),
        pltpu.VMEM((STAG,), jnp.int32),
        pltpu.VMEM((STAG,), jnp.int32),
        pltpu.VMEM((8, 128), jnp.int32),
        pltpu.SemaphoreType.DMA,
        pltpu.SemaphoreType.DMA,
    ],
)
def _sc_pass2(src_hbm, dst_hbm, lvl_hbm, cnt_hbm, srcc_hbm, dstc_hbm,
              summ_hbm, srcblk, dstblk, dlev_v, lvl_v, cnts_v, sstag, dstag,
              sblk, sem, osem):
    c = lax.axis_index("c")
    s = lax.axis_index("s")
    wid = s * NC + c
    woff = pl.multiple_of(wid * EPW, 8)
    pltpu.sync_copy(src_hbm.at[pl.ds(woff, EPW)], srcblk)
    pltpu.sync_copy(dst_hbm.at[pl.ds(woff, EPW)], dstblk)
    pltpu.sync_copy(lvl_hbm, lvl_v)
    pltpu.sync_copy(cnt_hbm, cnts_v)

    def _pg(g, _):
        d16 = dstblk[pl.ds(g * 16, 16)]
        dlev_v[pl.ds(g * 16, 16)] = plsc.load_gather(lvl_v, [d16])
        return 0
    lax.fori_loop(0, NGRP, _pg, 0)

    base, total, mine = _bucket_layout(cnts_v, wid)
    slot = base + mine

    @pl.when(wid == 0)
    def _pub():
        for r in range(8):
            for q in range(8):
                sblk[r, pl.ds(q * 16, 16)] = jnp.zeros((16,), jnp.int32)
        sblk[0, pl.ds(0, 16)] = base
        sblk[1, pl.ds(0, 16)] = total

        def _ng(g, nc):
            lv16 = lvl_v[pl.ds(g * 16, 16)]
            for l in range(8):
                pc = plsc.all_reduce_population_count(lv16 == l)
                nc = nc + jnp.where(_iota16() == l, pc, 0)
            return nc
        ncnt = lax.fori_loop(0, N // 16, _ng, jnp.zeros((16,), jnp.int32))
        sblk[2, pl.ds(0, 16)] = plsc.cumsum(ncnt) - ncnt
        sblk[3, pl.ds(0, 16)] = ncnt
        pltpu.sync_copy(sblk, summ_hbm)

    for l in range(1, 8):
        def _grp(g, ptr):
            s16 = srcblk[pl.ds(g * 16, 16)]
            d16 = dstblk[pl.ds(g * 16, 16)]
            m = dlev_v[pl.ds(g * 16, 16)] == l
            mi = m.astype(jnp.int32)
            idx = (plsc.cumsum(mi) - mi) + ptr
            plsc.store_scatter(sstag, [idx], s16, mask=m)
            plsc.store_scatter(dstag, [idx], d16, mask=m)
            return ptr + jnp.sum(mi)
        ptr = lax.fori_loop(0, NGRP, _grp, jnp.zeros((), jnp.int32))

        ones16 = jnp.zeros((16,), jnp.int32) + 1
        for g in range(CHUNK // 16):
            pidx = _iota16() + (ptr + g * 16)
            plsc.store_scatter(sstag, [pidx], jnp.zeros((16,), jnp.int32),
                               mask=ones16 == 1)
            plsc.store_scatter(dstag, [pidx],
                               jnp.zeros((16,), jnp.int32) + TRASH,
                               mask=ones16 == 1)

        myslot = _lane((_iota16() == l).astype(jnp.int32), slot)
        nch = (ptr + CHUNK - 1) // CHUNK

        def _out(j, _):
            o = pl.multiple_of(j * CHUNK, 8)
            go = pl.multiple_of(myslot + o, 8)
            pltpu.async_copy(sstag.at[pl.ds(o, CHUNK)],
                             srcc_hbm.at[pl.ds(go, CHUNK)], osem)
            pltpu.async_copy(dstag.at[pl.ds(o, CHUNK)],
                             dstc_hbm.at[pl.ds(go, CHUNK)], osem)
            return 0
        lax.fori_loop(0, nch, _out, 0)

        def _dr(j, _):
            pltpu.make_async_copy(sstag.at[pl.ds(0, CHUNK)],
                                  srcc_hbm.at[pl.ds(0, CHUNK)], osem).wait()
            pltpu.make_async_copy(dstag.at[pl.ds(0, CHUNK)],
                                  dstc_hbm.at[pl.ds(0, CHUNK)], osem).wait()
            return 0
        lax.fori_loop(0, nch, _dr, 0)


def _make_agg(lv):
    @functools.partial(
        pl.kernel,
        out_type=jax.ShapeDtypeStruct((NC, NP, HID), jnp.float32),
        mesh=_mesh,
        compiler_params=_params,
        scratch_types=[
            pltpu.VMEM((8, 128), jnp.int32),
            pltpu.VMEM((FAST * CHUNK,), jnp.int32),
            pltpu.VMEM((FAST * CHUNK,), jnp.int32),
            pltpu.VMEM((FAST, CHUNK), jnp.int32),
            pltpu.VMEM((CHUNK,), jnp.int32),
            pltpu.VMEM((CHUNK,), jnp.int32),
            pltpu.VMEM((CHUNK, HID), jnp.float32),
            pltpu.VMEM((CHUNK, HID), jnp.float32),
            pltpu.VMEM((CHUNK, HID), jnp.float32),
            pltpu.VMEM((CHUNK, HID), jnp.float32),
            pltpu.VMEM((ZU, HID), jnp.float32),
            pltpu.VMEM_SHARED((NP, HID), jnp.float32),
            pltpu.SemaphoreType.DMA,
            pltpu.SemaphoreType.DMA,
        ],
    )
    def _agg(hext_hbm, srcc_hbm, dstc_hbm, summ_hbm, zrs_hbm, out_hbm,
             summ_v, spre, dpre, dpre2, ssv, dsv, r00, r01, r10, r11,
             zero_v, acc_sh, gsem, qsem):
        c = lax.axis_index("c")
        s = lax.axis_index("s")
        wid = s * NC + c
        pltpu.sync_copy(summ_hbm, summ_v)
        pltpu.sync_copy(zrs_hbm, zero_v)
        oh = (_iota16() == lv).astype(jnp.int32)

        base_l = _lane(oh, summ_v[0, pl.ds(0, 16)])
        nch_l = _lane(oh, summ_v[1, pl.ds(0, 16)]) // CHUNK
        nb_l = _lane(oh, summ_v[2, pl.ds(0, 16)])
        ncn_l = _lane(oh, summ_v[3, pl.ds(0, 16)])

        st = (nch_l + NW - 1) // NW
        beg = wid * st
        T = jnp.maximum(0, jnp.minimum(st, nch_l - beg))
        poff = pl.multiple_of(base_l + beg * CHUNK, 8)
        pltpu.sync_copy(srcc_hbm.at[pl.ds(poff, FAST * CHUNK)], spre)
        pltpu.sync_copy(dstc_hbm.at[pl.ds(poff, FAST * CHUNK)], dpre)

        def _cp(i, _):
            for j in range(CHUNK // 16):
                dpre2[i, pl.ds(j * 16, 16)] = dpre[pl.ds(i * CHUNK + j * 16, 16)]
            return 0
        lax.fori_loop(0, FAST, _cp, 0)

        a0 = (nb_l // 8) * 8
        e0 = ((nb_l + ncn_l + 7) // 8) * 8
        nu = (e0 - a0 + ZU - 1) // ZU
        zt = (nu - s + NS - 1) // NS

        def _zcpy(i, _):
            off = pl.multiple_of(a0 + (s + i * NS) * ZU, 8)
            pltpu.sync_copy(zero_v, acc_sh.at[pl.ds(off, ZU)])
            return 0
        lax.fori_loop(0, zt, _zcpy, 0)
        plsc.subcore_barrier()

        rows = ((r00, r01), (r10, r11))
        TF = jnp.minimum(T, FAST)
        nwv = (TF + KW - 1) // KW

        def _s_drain():
            pltpu.make_async_copy(r00, acc_sh.at[dpre2.at[0]], qsem).wait()

        def _wave(w, _):
            sigma = w % 2
            for sg in range(2):
                @pl.when(sigma == sg)
                def _do():
                    rset = rows[sg]
                    for k in range(KW):
                        cid2 = (w - 2) * KW + k

                        @pl.when((w >= 2) & (cid2 < TF))
                        def _ds():
                            _s_drain()
                    for k in range(KW):
                        cid = w * KW + k

                        @pl.when(cid < TF)
                        def _fg():
                            idx = spre.at[pl.ds(cid * CHUNK, CHUNK)]
                            pltpu.async_copy(hext_hbm.at[idx], rset[k], gsem)
                    for k in range(KW):
                        cid = w * KW + k

                        @pl.when(cid < TF)
                        def _dg():
                            idx = spre.at[pl.ds(cid * CHUNK, CHUNK)]
                            pltpu.make_async_copy(
                                hext_hbm.at[idx], rset[k], gsem).wait()
                    for k in range(KW):
                        cid = w * KW + k

                        @pl.when(cid < TF)
                        def _fs():
                            pltpu.async_copy(rset[k], acc_sh.at[dpre2.at[cid]],
                                             qsem, add=True)
            return 0
        lax.fori_loop(0, nwv, _wave, 0)

        drained = jnp.maximum(0, (nwv - 2)) * KW
        rest = TF - drained
        for i in range(2 * KW):
            @pl.when(i < rest)
            def _dt():
                _s_drain()

        def _chunk(j, _):
            off = pl.multiple_of(base_l + (beg + FAST + j) * CHUNK, 8)
            pltpu.sync_copy(srcc_hbm.at[pl.ds(off, CHUNK)], ssv)
            pltpu.sync_copy(dstc_hbm.at[pl.ds(off, CHUNK)], dsv)
            pltpu.async_copy(hext_hbm.at[ssv], r00, gsem).wait()
            pltpu.sync_copy(r00, acc_sh.at[dsv], add=True)
            return 0
        lax.fori_loop(0, jnp.maximum(0, T - FAST), _chunk, 0)
        plsc.subcore_barrier()

        def _dcpy(i, _):
            off = pl.multiple_of(a0 + (s + i * NS) * ZU, 8)
            pltpu.sync_copy(acc_sh.at[pl.ds(off, ZU)],
                            out_hbm.at[c, pl.ds(off, ZU)])
            return 0
        lax.fori_loop(0, zt, _dcpy, 0)
    return _agg


_AGGS = {l: _make_agg(l) for l in range(1, NLVL)}


_RB = 1024
_NBLK = NP // _RB


def _leaky(x):
    return jnp.where(x >= 0, x, 0.1 * x)


def _tc_init_body(feat, delay, lvl, Wpi1, bpi1, Wpi2, bpi2, Ws1, bs1, Ws2,
                  bs2, h_out, hself_out):
    hs = jnp.dot(feat[...], Ws1[...], preferred_element_type=jnp.float32)
    hs = _leaky(hs + bs1[...])
    hs = jnp.dot(hs, Ws2[...], preferred_element_type=jnp.float32) + bs2[...]
    hself_out[...] = hs

    hp = delay[...] * Wpi1[...]
    hp = _leaky(hp + bpi1[...])
    hp = jnp.dot(hp, Wpi2[...], preferred_element_type=jnp.float32) + bpi2[...]
    h_out[...] = jnp.where(lvl[...] == 0, hp, 0.0)


def _tc_init(feat, delay, lvl2d, Wpi1, bpi1, Wpi2, bpi2, Ws1, bs1, Ws2, bs2):
    full = lambda shape: pl.BlockSpec(shape, lambda i: (0, 0))
    row = lambda w: pl.BlockSpec((_RB, w), lambda i: (i, 0))
    return pl.pallas_call(
        _tc_init_body,
        grid=(_NBLK,),
        in_specs=[row(HID), row(1), row(1),
                  full((1, 64)), full((1, 64)), full((64, HID)), full((1, HID)),
                  full((HID, 64)), full((1, 64)), full((64, HID)), full((1, HID))],
        out_specs=[row(HID), row(HID)],
        out_shape=[jax.ShapeDtypeStruct((NP, HID), jnp.float32),
                   jax.ShapeDtypeStruct((NP, HID), jnp.float32)],
    )(feat, delay, lvl2d, Wpi1, bpi1, Wpi2, bpi2, Ws1, bs1, Ws2, bs2)


def _tc_level_body(lref, accA, accB, degA, degB, hself, h_in, lvl,
                   ispo, Wn1, bn1, Wn2, bn2, h_out):
    lv = lref[0, 0]
    deg = jnp.maximum(degA[...] + degB[...], 1.0)
    neigh = (accA[...] + accB[...]) / deg
    hid = jnp.dot(neigh, Wn1[...], preferred_element_type=jnp.float32)
    hid = _leaky(hid + bn1[...])
    out = jnp.dot(hid, Wn2[...], preferred_element_type=jnp.float32) + bn2[...]
    out = out + hself[...]
    out = jnp.where(ispo[...] != 1, jnp.maximum(out, 0.0), out)
    h_out[...] = jnp.where(lvl[...] == lv, out, h_in[...])


def _tc_level(lval, accA, accB, degA, degB, hself, h, lvl2d, ispo,
              Wn1, bn1, Wn2, bn2):
    full = lambda shape: pl.BlockSpec(shape, lambda i: (0, 0))
    row = lambda w: pl.BlockSpec((_RB, w), lambda i: (i, 0))
    return pl.pallas_call(
        _tc_level_body,
        grid=(_NBLK,),
        in_specs=[pl.BlockSpec(memory_space=pltpu.SMEM),
                  row(HID), row(HID), row(1), row(1), row(HID), row(HID),
                  row(1), row(1),
                  full((HID, 64)), full((1, 64)), full((64, HID)), full((1, HID))],
        out_specs=row(HID),
        out_shape=jax.ShapeDtypeStruct((NP, HID), jnp.float32),
    )(lval, accA, accB, degA, degB, hself, h, lvl2d, ispo,
      Wn1, bn1, Wn2, bn2)


def kernel(feat, delay, is_po, edge_index, level_ids, Wpi1, bpi1, Wpi2, bpi2,
           Ws1, bs1, Ws2, bs2, Wn1, bn1, Wn2, bn2):
    src = edge_index[0]
    dst = edge_index[1]
    pad = NP - N
    featp = jnp.pad(feat, ((0, pad), (0, 0)))
    delayp = jnp.pad(delay, ((0, pad), (0, 0)))
    ispop = jnp.pad(is_po, ((0, pad), (0, 0)))
    lvlp = jnp.pad(level_ids, (0, pad), constant_values=99)[:, None]

    h, h_self = _tc_init(featp, delayp, lvlp,
                         Wpi1, bpi1[None, :], Wpi2, bpi2[None, :],
                         Ws1, bs1[None, :], Ws2, bs2[None, :])

    ones_tab = jnp.ones((DCH, HID), jnp.float32)
    zrs_tab = jnp.zeros((ZU, HID), jnp.float32)
    degp, cnts = _sc_degp1(dst.reshape(NW, NKCH, DCH), level_ids,
                           ones_tab, zrs_tab)
    degA = degp[0, :, 0:1]
    degB = degp[1, :, 0:1]
    srcc, dstc, summ = _sc_pass2(src, dst, level_ids, cnts)

    bn1r = bn1[None, :]
    bn2r = bn2[None, :]
    for l in range(1, NLVL):
        acc = _AGGS[l](h, srcc, dstc, summ, zrs_tab)
        lval = jnp.full((1, 1), l, dtype=jnp.int32)
        h = _tc_level(lval, acc[0], acc[1], degA, degB, h_self, h,
                      lvlp, ispop, Wn1, bn1r, Wn2, bn2r)
    return h[:N]


# speedup vs baseline: 1.3259x; 1.0119x over previous
import functools

import jax
import jax.numpy as jnp
from jax import lax
from jax.experimental import pallas as pl
from jax.experimental.pallas import tpu as pltpu
from jax.experimental.pallas import tpu_sc as plsc

N = 10000
E = 320000
HID = 128
NLVL = 8

NC = 2
NS = 16
NW = NC * NS
EPW = E // NW
NGRP = EPW // 16
CHUNK = 64
DCH = 80
NKCH = EPW // DCH
WAVE = 25
NP = 10240
RPS = NP // NS
ZU = 32
TRASH = NP - 8
EP = E + CHUNK * 256 + 8192
FAST = 32
KW = 2
STAG = EPW + CHUNK

_mesh = plsc.VectorSubcoreMesh(
    core_axis_name="c", subcore_axis_name="s", num_cores=NC, num_subcores=NS)
_params = pltpu.CompilerParams(needs_layout_passes=False)


def _iota16():
    return lax.iota(jnp.int32, 16)


def _lane(oh, v16):
    return jnp.sum(jnp.where(oh == 1, v16, 0))


@functools.partial(
    pl.kernel,
    out_type=[jax.ShapeDtypeStruct((NC, NP, HID), jnp.float32),
              jax.ShapeDtypeStruct((NW, 8, 128), jnp.int32)],
    mesh=_mesh,
    compiler_params=_params,
    scratch_types=[
        pltpu.VMEM((NKCH, DCH), jnp.int32),
        pltpu.VMEM((DCH, HID), jnp.float32),
        pltpu.VMEM((ZU, HID), jnp.float32),
        pltpu.VMEM((N,), jnp.int32),
        pltpu.VMEM((8, 128), jnp.int32),
        pltpu.VMEM_SHARED((NP, HID), jnp.float32),
        pltpu.SemaphoreType.DMA,
        pltpu.SemaphoreType.DMA,
    ],
)
def _sc_degp1(dst2_hbm, lvl_hbm, ones_hbm, zrs_hbm, deg_hbm, cnt_hbm,
              dstblk, ones_v, zero_v, lvl_v, cblk, acc_sh, sem, dsem):
    c = lax.axis_index("c")
    s = lax.axis_index("s")
    wid = s * NC + c
    pltpu.sync_copy(dst2_hbm.at[wid], dstblk)
    pltpu.sync_copy(lvl_hbm, lvl_v)
    pltpu.sync_copy(ones_hbm, ones_v)
    pltpu.sync_copy(zrs_hbm, zero_v)

    for r in range(8):
        for q in range(8):
            cblk[r, pl.ds(q * 16, 16)] = jnp.zeros((16,), jnp.int32)

    def _zcpy(i, _):
        off = pl.multiple_of(s * RPS + i * ZU, 8)
        pltpu.sync_copy(zero_v, acc_sh.at[pl.ds(off, ZU)])
        return 0
    lax.fori_loop(0, RPS // ZU, _zcpy, 0)
    plsc.subcore_barrier()

    def _wave(wv, cnt):
        def _fire(k, _):
            pltpu.async_copy(ones_v, acc_sh.at[dstblk.at[wv * WAVE + k]],
                             dsem, add=True)
            return 0
        lax.fori_loop(0, WAVE, _fire, 0)

        def _grp(g, cn):
            g2 = wv * (NGRP // 5) + g
            d16 = dstblk[g2 // 5, pl.ds((g2 % 5) * 16, 16)]
            dlev = plsc.load_gather(lvl_v, [d16])
            for l in range(1, 8):
                pc = plsc.all_reduce_population_count(dlev == l)
                cn = cn + jnp.where(_iota16() == l, pc, 0)
            return cn
        cnt = lax.fori_loop(0, NGRP // 5, _grp, cnt)

        def _drain(k, _):
            pltpu.make_async_copy(ones_v, acc_sh.at[dstblk.at[0]],
                                  dsem).wait()
            return 0
        lax.fori_loop(0, WAVE, _drain, 0)
        return cnt
    cnt = lax.fori_loop(0, NKCH // WAVE, _wave, jnp.zeros((16,), jnp.int32))

    cblk[0, pl.ds(0, 16)] = cnt
    pltpu.sync_copy(cblk, cnt_hbm.at[wid])
    plsc.subcore_barrier()

    off = pl.multiple_of(s * RPS, 8)
    pltpu.sync_copy(acc_sh.at[pl.ds(off, RPS)],
                    deg_hbm.at[c, pl.ds(off, RPS)])


def _bucket_layout(cnts_v, wid):
    total = jnp.zeros((16,), jnp.int32)
    mine = jnp.zeros((16,), jnp.int32)
    widv = jnp.zeros((16,), jnp.int32) + wid
    for wo in range(NW):
        row = cnts_v[wo, 0, pl.ds(0, 16)]
        pcw = ((row + (CHUNK - 1)) // CHUNK) * CHUNK
        total = total + pcw
        mine = mine + jnp.where(widv > wo, pcw, 0)
    base = plsc.cumsum(total) - total
    return base, total, mine


@functools.partial(
    pl.kernel,
    out_type=[jax.ShapeDtypeStruct((EP,), jnp.int32),
              jax.ShapeDtypeStruct((EP,), jnp.int32),
              jax.ShapeDtypeStruct((8, 128), jnp.int32)],
    mesh=_mesh,
    compiler_params=_params,
    scratch_types=[
        pltpu.VMEM((EPW,), jnp.int32),
        pltpu.VMEM((EPW,), jnp.int32),
        pltpu.VMEM((EPW,), jnp.int32),
        pltpu.VMEM((N,), jnp.int32),
        pltpu.VMEM((NW, 8, 128), jnp.int32),
        pltpu.VMEM((STAG,), jnp.int32),
        pltpu.VMEM((STAG,), jnp.int32),
        pltpu.VMEM((8, 128), jnp.int32),
        pltpu.SemaphoreType.DMA,
        pltpu.SemaphoreType.DMA,
    ],
)
def _sc_pass2(src_hbm, dst_hbm, lvl_hbm, cnt_hbm, srcc_hbm, dstc_hbm,
              summ_hbm, srcblk, dstblk, dlev_v, lvl_v, cnts_v, sstag, dstag,
              sblk, sem, osem):
    c = lax.axis_index("c")
    s = lax.axis_index("s")
    wid = s * NC + c
    woff = pl.multiple_of(wid * EPW, 8)
    pltpu.sync_copy(src_hbm.at[pl.ds(woff, EPW)], srcblk)
    pltpu.sync_copy(dst_hbm.at[pl.ds(woff, EPW)], dstblk)
    pltpu.sync_copy(lvl_hbm, lvl_v)
    pltpu.sync_copy(cnt_hbm, cnts_v)

    def _pg(g, _):
        d16 = dstblk[pl.ds(g * 16, 16)]
        dlev_v[pl.ds(g * 16, 16)] = plsc.load_gather(lvl_v, [d16])
        return 0
    lax.fori_loop(0, NGRP, _pg, 0)

    base, total, mine = _bucket_layout(cnts_v, wid)
    slot = base + mine

    @pl.when(wid == 0)
    def _pub():
        for r in range(8):
            for q in range(8):
                sblk[r, pl.ds(q * 16, 16)] = jnp.zeros((16,), jnp.int32)
        sblk[0, pl.ds(0, 16)] = base
        sblk[1, pl.ds(0, 16)] = total

        def _ng(g, nc):
            lv16 = lvl_v[pl.ds(g * 16, 16)]
            for l in range(8):
                pc = plsc.all_reduce_population_count(lv16 == l)
                nc = nc + jnp.where(_iota16() == l, pc, 0)
            return nc
        ncnt = lax.fori_loop(0, N // 16, _ng, jnp.zeros((16,), jnp.int32))
        sblk[2, pl.ds(0, 16)] = plsc.cumsum(ncnt) - ncnt
        sblk[3, pl.ds(0, 16)] = ncnt
        pltpu.sync_copy(sblk, summ_hbm)

    for l in range(1, 8):
        def _grp(g, ptr):
            s16 = srcblk[pl.ds(g * 16, 16)]
            d16 = dstblk[pl.ds(g * 16, 16)]
            m = dlev_v[pl.ds(g * 16, 16)] == l
            mi = m.astype(jnp.int32)
            idx = (plsc.cumsum(mi) - mi) + ptr
            plsc.store_scatter(sstag, [idx], s16, mask=m)
            plsc.store_scatter(dstag, [idx], d16, mask=m)
            return ptr + jnp.sum(mi)
        ptr = lax.fori_loop(0, NGRP, _grp, jnp.zeros((), jnp.int32))

        ones16 = jnp.zeros((16,), jnp.int32) + 1
        for g in range(CHUNK // 16):
            pidx = _iota16() + (ptr + g * 16)
            plsc.store_scatter(sstag, [pidx], jnp.zeros((16,), jnp.int32),
                               mask=ones16 == 1)
            plsc.store_scatter(dstag, [pidx],
                               jnp.zeros((16,), jnp.int32) + TRASH,
                               mask=ones16 == 1)

        myslot = _lane((_iota16() == l).astype(jnp.int32), slot)
        nch = (ptr + CHUNK - 1) // CHUNK

        def _out(j, _):
            o = pl.multiple_of(j * CHUNK, 8)
            go = pl.multiple_of(myslot + o, 8)
            pltpu.async_copy(sstag.at[pl.ds(o, CHUNK)],
                             srcc_hbm.at[pl.ds(go, CHUNK)], osem)
            pltpu.async_copy(dstag.at[pl.ds(o, CHUNK)],
                             dstc_hbm.at[pl.ds(go, CHUNK)], osem)
            return 0
        lax.fori_loop(0, nch, _out, 0)

        def _dr(j, _):
            pltpu.make_async_copy(sstag.at[pl.ds(0, CHUNK)],
                                  srcc_hbm.at[pl.ds(0, CHUNK)], osem).wait()
            pltpu.make_async_copy(dstag.at[pl.ds(0, CHUNK)],
                                  dstc_hbm.at[pl.ds(0, CHUNK)], osem).wait()
            return 0
        lax.fori_loop(0, nch, _dr, 0)


def _make_agg(lv):
    @functools.partial(
        pl.kernel,
        out_type=jax.ShapeDtypeStruct((NC, NP, HID), jnp.float32),
        mesh=_mesh,
        compiler_params=_params,
        scratch_types=[
            pltpu.VMEM((8, 128), jnp.int32),
            pltpu.VMEM((FAST * CHUNK,), jnp.int32),
            pltpu.VMEM((FAST * CHUNK,), jnp.int32),
            pltpu.VMEM((FAST, CHUNK), jnp.int32),
            pltpu.VMEM((CHUNK,), jnp.int32),
            pltpu.VMEM((CHUNK,), jnp.int32),
            pltpu.VMEM((CHUNK, HID), jnp.float32),
            pltpu.VMEM((CHUNK, HID), jnp.float32),
            pltpu.VMEM((CHUNK, HID), jnp.float32),
            pltpu.VMEM((CHUNK, HID), jnp.float32),
            pltpu.VMEM((ZU, HID), jnp.float32),
            pltpu.VMEM_SHARED((NP, HID), jnp.float32),
            pltpu.SemaphoreType.DMA,
            pltpu.SemaphoreType.DMA,
        ],
    )
    def _agg(hext_hbm, srcc_hbm, dstc_hbm, summ_hbm, zrs_hbm, out_hbm,
             summ_v, spre, dpre, dpre2, ssv, dsv, r00, r01, r10, r11,
             zero_v, acc_sh, gsem, qsem):
        c = lax.axis_index("c")
        s = lax.axis_index("s")
        wid = s * NC + c
        pltpu.sync_copy(summ_hbm, summ_v)
        pltpu.sync_copy(zrs_hbm, zero_v)
        oh = (_iota16() == lv).astype(jnp.int32)

        base_l = _lane(oh, summ_v[0, pl.ds(0, 16)])
        nch_l = _lane(oh, summ_v[1, pl.ds(0, 16)]) // CHUNK
        nb_l = _lane(oh, summ_v[2, pl.ds(0, 16)])
        ncn_l = _lane(oh, summ_v[3, pl.ds(0, 16)])

        st = (nch_l + NW - 1) // NW
        beg = wid * st
        T = jnp.maximum(0, jnp.minimum(st, nch_l - beg))
        poff = pl.multiple_of(base_l + beg * CHUNK, 8)
        pltpu.sync_copy(srcc_hbm.at[pl.ds(poff, FAST * CHUNK)], spre)
        pltpu.sync_copy(dstc_hbm.at[pl.ds(poff, FAST * CHUNK)], dpre)

        def _cp(i, _):
            for j in range(CHUNK // 16):
                dpre2[i, pl.ds(j * 16, 16)] = dpre[pl.ds(i * CHUNK + j * 16, 16)]
            return 0
        lax.fori_loop(0, FAST, _cp, 0)

        a0 = (nb_l // 8) * 8
        e0 = ((nb_l + ncn_l + 7) // 8) * 8
        nu = (e0 - a0 + ZU - 1) // ZU
        zt = (nu - s + NS - 1) // NS

        def _zcpy(i, _):
            off = pl.multiple_of(a0 + (s + i * NS) * ZU, 8)
            pltpu.sync_copy(zero_v, acc_sh.at[pl.ds(off, ZU)])
            return 0
        lax.fori_loop(0, zt, _zcpy, 0)
        plsc.subcore_barrier()

        rows = ((r00, r01), (r10, r11))
        TF = jnp.minimum(T, FAST)
        nwv = (TF + KW - 1) // KW

        def _s_drain():
            pltpu.make_async_copy(r00, acc_sh.at[dpre2.at[0]], qsem).wait()

        def _g_drain():
            pltpu.make_async_copy(
                hext_hbm.at[spre.at[pl.ds(0, CHUNK)]], r00, gsem).wait()

        def _wave(w, _):
            sigma = w % 2
            for k in range(KW):
                cid2 = (w - 2) * KW + k

                @pl.when((w >= 2) & (cid2 < TF))
                def _ds():
                    _s_drain()
            for sg in range(2):
                @pl.when(sigma == sg)
                def _do():
                    cur = rows[sg]
                    prv = rows[1 - sg]
                    for k in range(KW):
                        cid = w * KW + k

                        @pl.when(cid < TF)
                        def _fg():
                            idx = spre.at[pl.ds(cid * CHUNK, CHUNK)]
                            pltpu.async_copy(hext_hbm.at[idx], cur[k], gsem)
                    for k in range(KW):
                        cidp = (w - 1) * KW + k

                        @pl.when((w >= 1) & (cidp < TF))
                        def _dgfs():
                            _g_drain()
                            pltpu.async_copy(prv[k],
                                             acc_sh.at[dpre2.at[cidp]],
                                             qsem, add=True)
            return 0
        lax.fori_loop(0, nwv, _wave, 0)

        for sg in range(2):
            @pl.when((nwv >= 1) & ((nwv - 1) % 2 == sg))
            def _tl():
                cur = rows[sg]
                for k in range(KW):
                    cid = (nwv - 1) * KW + k

                    @pl.when(cid < TF)
                    def _t1():
                        _g_drain()
                        pltpu.async_copy(cur[k], acc_sh.at[dpre2.at[cid]],
                                        qsem, add=True)

        drained = jnp.maximum(0, (nwv - 2)) * KW
        rest = TF - drained
        for i in range(2 * KW):
            @pl.when(i < rest)
            def _dt():
                _s_drain()

        def _chunk(j, _):
            off = pl.multiple_of(base_l + (beg + FAST + j) * CHUNK, 8)
            pltpu.sync_copy(srcc_hbm.at[pl.ds(off, CHUNK)], ssv)
            pltpu.sync_copy(dstc_hbm.at[pl.ds(off, CHUNK)], dsv)
            pltpu.async_copy(hext_hbm.at[ssv], r00, gsem).wait()
            pltpu.sync_copy(r00, acc_sh.at[dsv], add=True)
            return 0
        lax.fori_loop(0, jnp.maximum(0, T - FAST), _chunk, 0)
        plsc.subcore_barrier()

        def _dcpy(i, _):
            off = pl.multiple_of(a0 + (s + i * NS) * ZU, 8)
            pltpu.sync_copy(acc_sh.at[pl.ds(off, ZU)],
                            out_hbm.at[c, pl.ds(off, ZU)])
            return 0
        lax.fori_loop(0, zt, _dcpy, 0)
    return _agg


_AGGS = {l: _make_agg(l) for l in range(1, NLVL)}


_RB = 1024
_NBLK = NP // _RB


def _leaky(x):
    return jnp.where(x >= 0, x, 0.1 * x)


def _tc_init_body(feat, delay, lvl, Wpi1, bpi1, Wpi2, bpi2, Ws1, bs1, Ws2,
                  bs2, h_out, hself_out):
    hs = jnp.dot(feat[...], Ws1[...], preferred_element_type=jnp.float32)
    hs = _leaky(hs + bs1[...])
    hs = jnp.dot(hs, Ws2[...], preferred_element_type=jnp.float32) + bs2[...]
    hself_out[...] = hs

    hp = delay[...] * Wpi1[...]
    hp = _leaky(hp + bpi1[...])
    hp = jnp.dot(hp, Wpi2[...], preferred_element_type=jnp.float32) + bpi2[...]
    h_out[...] = jnp.where(lvl[...] == 0, hp, 0.0)


def _tc_init(feat, delay, lvl2d, Wpi1, bpi1, Wpi2, bpi2, Ws1, bs1, Ws2, bs2):
    full = lambda shape: pl.BlockSpec(shape, lambda i: (0, 0))
    row = lambda w: pl.BlockSpec((_RB, w), lambda i: (i, 0))
    return pl.pallas_call(
        _tc_init_body,
        grid=(_NBLK,),
        in_specs=[row(HID), row(1), row(1),
                  full((1, 64)), full((1, 64)), full((64, HID)), full((1, HID)),
                  full((HID, 64)), full((1, 64)), full((64, HID)), full((1, HID))],
        out_specs=[row(HID), row(HID)],
        out_shape=[jax.ShapeDtypeStruct((NP, HID), jnp.float32),
                   jax.ShapeDtypeStruct((NP, HID), jnp.float32)],
    )(feat, delay, lvl2d, Wpi1, bpi1, Wpi2, bpi2, Ws1, bs1, Ws2, bs2)


def _tc_level_body(lref, accA, accB, degA, degB, hself, h_in, lvl,
                   ispo, Wn1, bn1, Wn2, bn2, h_out):
    lv = lref[0, 0]
    deg = jnp.maximum(degA[...] + degB[...], 1.0)
    neigh = (accA[...] + accB[...]) / deg
    hid = jnp.dot(neigh, Wn1[...], preferred_element_type=jnp.float32)
    hid = _leaky(hid + bn1[...])
    out = jnp.dot(hid, Wn2[...], preferred_element_type=jnp.float32) + bn2[...]
    out = out + hself[...]
    out = jnp.where(ispo[...] != 1, jnp.maximum(out, 0.0), out)
    h_out[...] = jnp.where(lvl[...] == lv, out, h_in[...])


def _tc_level(lval, accA, accB, degA, degB, hself, h, lvl2d, ispo,
              Wn1, bn1, Wn2, bn2):
    full = lambda shape: pl.BlockSpec(shape, lambda i: (0, 0))
    row = lambda w: pl.BlockSpec((_RB, w), lambda i: (i, 0))
    return pl.pallas_call(
        _tc_level_body,
        grid=(_NBLK,),
        in_specs=[pl.BlockSpec(memory_space=pltpu.SMEM),
                  row(HID), row(HID), row(1), row(1), row(HID), row(HID),
                  row(1), row(1),
                  full((HID, 64)), full((1, 64)), full((64, HID)), full((1, HID))],
        out_specs=row(HID),
        out_shape=jax.ShapeDtypeStruct((NP, HID), jnp.float32),
    )(lval, accA, accB, degA, degB, hself, h, lvl2d, ispo,
      Wn1, bn1, Wn2, bn2)


def kernel(feat, delay, is_po, edge_index, level_ids, Wpi1, bpi1, Wpi2, bpi2,
           Ws1, bs1, Ws2, bs2, Wn1, bn1, Wn2, bn2):
    src = edge_index[0]
    dst = edge_index[1]
    pad = NP - N
    featp = jnp.pad(feat, ((0, pad), (0, 0)))
    delayp = jnp.pad(delay, ((0, pad), (0, 0)))
    ispop = jnp.pad(is_po, ((0, pad), (0, 0)))
    lvlp = jnp.pad(level_ids, (0, pad), constant_values=99)[:, None]

    h, h_self = _tc_init(featp, delayp, lvlp,
                         Wpi1, bpi1[None, :], Wpi2, bpi2[None, :],
                         Ws1, bs1[None, :], Ws2, bs2[None, :])

    ones_tab = jnp.ones((DCH, HID), jnp.float32)
    zrs_tab = jnp.zeros((ZU, HID), jnp.float32)
    degp, cnts = _sc_degp1(dst.reshape(NW, NKCH, DCH), level_ids,
                           ones_tab, zrs_tab)
    degA = degp[0, :, 0:1]
    degB = degp[1, :, 0:1]
    srcc, dstc, summ = _sc_pass2(src, dst, level_ids, cnts)

    bn1r = bn1[None, :]
    bn2r = bn2[None, :]
    for l in range(1, NLVL):
        acc = _AGGS[l](h, srcc, dstc, summ, zrs_tab)
        lval = jnp.full((1, 1), l, dtype=jnp.int32)
        h = _tc_level(lval, acc[0], acc[1], degA, degB, h_self, h,
                      lvlp, ispop, Wn1, bn1r, Wn2, bn2r)
    return h[:N]
